# Initial kernel scaffold; baseline (speedup 1.0000x reference)
import jax
import jax.numpy as jnp
from jax.experimental import pallas as pl


def kernel(y, edge_index, norm_g, norm_b, Wl, bl, Wr, br, dln_g, dln_b, W1, b1, W2, b2):
    raise NotImplementedError("write your pallas kernel here")


# speedup vs baseline: 1.6715x; 1.6715x over previous
import jax
import jax.numpy as jnp
from jax import lax
from jax.experimental import pallas as pl
from jax.experimental.pallas import tpu as pltpu
from jax.experimental.pallas import tpu_sc as plsc

N_NODES = 10000
E = 320000
H = 128
N_OUT = 128
_EPS = 1e-5

NC = 2
NS = 16
L = 16

HP = H + L
N_PAD = 10240
NPT = N_PAD // NS
SUB = 80
CB = 80
K = 8
CH = CB * K
RB = 160
CHUNKS = E // CH

EB = 1280


def _gates_body(y_ref, ng_ref, nb_ref, wl_ref, bl_ref, wr_ref, br_ref, g_ref):
    y = y_ref[...]
    m = jnp.mean(y, axis=-1, keepdims=True)
    c = y - m
    v = jnp.mean(c * c, axis=-1, keepdims=True)
    yn = c * lax.rsqrt(v + _EPS) * ng_ref[...] + nb_ref[...]
    zl = jnp.dot(yn, wl_ref[...], preferred_element_type=jnp.float32) + bl_ref[...]
    zr = jnp.dot(yn, wr_ref[...], preferred_element_type=jnp.float32) + br_ref[...]
    ones = jnp.ones((EB, L), jnp.float32)
    g_ref[0, :, pl.ds(0, H)] = 1.0 / (1.0 + jnp.exp(-zl))
    g_ref[1, :, pl.ds(0, H)] = 1.0 / (1.0 + jnp.exp(-zr))
    g_ref[0, :, pl.ds(H, L)] = ones
    g_ref[1, :, pl.ds(H, L)] = ones


def _gates(y, ng, nb, wl, bl, wr, br):
    full = lambda shape: pl.BlockSpec(shape, lambda i: tuple(0 for _ in shape))
    return pl.pallas_call(
        _gates_body,
        grid=(E // EB,),
        in_specs=[
            pl.BlockSpec((EB, H), lambda i: (i, 0)),
            full((1, H)), full((1, H)),
            full((H, H)), full((1, H)),
            full((H, H)), full((1, H)),
        ],
        out_specs=pl.BlockSpec((2, EB, HP), lambda i: (0, i, 0)),
        out_shape=jax.ShapeDtypeStruct((2, E, HP), jnp.float32),
    )(y, ng, nb, wl, bl, wr, br)


def _sc_body(g_hbm, ei_hbm, h_hbm, acc, rows, idxb, nbuf, sem):
    cid = lax.axis_index("c")
    sid = lax.axis_index("s")
    node0 = sid * NPT
    n_my = jnp.where(sid < CHUNKS % NS, CHUNKS // NS + 1, CHUNKS // NS)
    zero16 = jnp.zeros((L,), jnp.float32)

    def _zrow(i, _):
        def _zcol(j, _):
            nbuf[i, pl.ds(j * L, L)] = zero16
            return 0
        return lax.fori_loop(0, HP // L, _zcol, 0)
    lax.fori_loop(0, SUB, _zrow, 0)

    for k in range(NPT // SUB):
        pltpu.sync_copy(nbuf, acc.at[pl.ds(node0 + k * SUB, SUB)])
    plsc.subcore_barrier()

    def _scatter_chunk(t, _):
        c = sid + NS * t
        pltpu.sync_copy(ei_hbm.at[1, pl.ds(c * K, K)], idxb)
        for s2 in range(CH // RB):
            pltpu.sync_copy(g_hbm.at[cid, pl.ds(c * CH + s2 * RB, RB)], rows)
            for j2 in range(RB // CB):
                jrow = s2 * (RB // CB) + j2
                pltpu.sync_copy(rows.at[pl.ds(j2 * CB, CB)],
                                acc.at[idxb.at[jrow]], add=True)
        return 0
    lax.fori_loop(0, n_my, _scatter_chunk, 0)
    plsc.subcore_barrier()

    for k in range(NPT // SUB):
        base = node0 + k * SUB
        pltpu.sync_copy(acc.at[pl.ds(base, SUB)], nbuf)

        def _nrow(n, _):
            deg = nbuf[n, pl.ds(H, L)]
            inv = 1.0 / jnp.maximum(deg, 1.0)
            for j in range(H // L):
                nbuf[n, pl.ds(j * L, L)] = nbuf[n, pl.ds(j * L, L)] * inv
            return 0
        lax.fori_loop(0, SUB, _nrow, 0)
        pltpu.sync_copy(nbuf, acc.at[pl.ds(base, SUB)])
    plsc.subcore_barrier()

    def _gather_chunk(t, _):
        c = sid + NS * t
        pltpu.sync_copy(ei_hbm.at[cid, pl.ds(c * K, K)], idxb)
        for s2 in range(CH // RB):
            descs = [
                pltpu.async_copy(acc.at[idxb.at[s2 * (RB // CB) + j2]],
                                 rows.at[pl.ds(j2 * CB, CB)], sem)
                for j2 in range(RB // CB)
            ]
            for d in descs:
                d.wait()
            pltpu.sync_copy(rows, h_hbm.at[cid, pl.ds(c * CH + s2 * RB, RB)])
        return 0
    lax.fori_loop(0, n_my, _gather_chunk, 0)


def _sc_aggregate(g, ei3):
    mesh = plsc.VectorSubcoreMesh(core_axis_name="c", subcore_axis_name="s",
                                  num_cores=NC, num_subcores=NS)
    return pl.kernel(
        _sc_body,
        out_type=jax.ShapeDtypeStruct((2, E, HP), jnp.float32),
        mesh=mesh,
        scratch_types=[
            pltpu.VMEM_SHARED((N_PAD, HP), jnp.float32),
            pltpu.VMEM((RB, HP), jnp.float32),
            pltpu.VMEM((K, CB), jnp.int32),
            pltpu.VMEM((SUB, HP), jnp.float32),
            pltpu.SemaphoreType.DMA,
        ],
        compiler_params=pltpu.CompilerParams(needs_layout_passes=False,
                                             use_tc_tiling_on_sc=False),
    )(g, ei3)


def _mlp_body(h_ref, dg_ref, db_ref, w1l_ref, w1r_ref, b1_ref, w2_ref,
              b2_ref, o_ref):
    hl = h_ref[0, :, pl.ds(0, H)]
    hr = h_ref[1, :, pl.ds(0, H)]
    m = (jnp.sum(hl, axis=-1, keepdims=True)
         + jnp.sum(hr, axis=-1, keepdims=True)) / (2.0 * H)
    cl = hl - m
    cr = hr - m
    v = (jnp.sum(cl * cl, axis=-1, keepdims=True)
         + jnp.sum(cr * cr, axis=-1, keepdims=True)) / (2.0 * H)
    rstd = lax.rsqrt(v + _EPS)
    lnl = cl * rstd * dg_ref[0] + db_ref[0]
    lnr = cr * rstd * dg_ref[1] + db_ref[1]
    z = (jnp.dot(lnl, w1l_ref[...], preferred_element_type=jnp.float32)
         + jnp.dot(lnr, w1r_ref[...], preferred_element_type=jnp.float32)
         + b1_ref[...])
    e = jnp.exp(jnp.minimum(z, 0.0)) - 1.0
    a = jnp.where(z > 0.0, z, e)
    o_ref[...] = (jnp.dot(a, w2_ref[...], preferred_element_type=jnp.float32)
                  + b2_ref[...])


def _mlp(h, dg, db, w1l, w1r, b1, w2, b2):
    full = lambda shape: pl.BlockSpec(shape, lambda i: tuple(0 for _ in shape))
    return pl.pallas_call(
        _mlp_body,
        grid=(E // EB,),
        in_specs=[
            pl.BlockSpec((2, EB, HP), lambda i: (0, i, 0)),
            full((2, H)), full((2, H)),
            full((H, H)), full((H, H)), full((1, H)),
            full((H, N_OUT)), full((1, N_OUT)),
        ],
        out_specs=pl.BlockSpec((EB, N_OUT), lambda i: (i, 0)),
        out_shape=jax.ShapeDtypeStruct((E, N_OUT), jnp.float32),
    )(h, dg, db, w1l, w1r, b1, w2, b2)


def kernel(y, edge_index, norm_g, norm_b, Wl, bl, Wr, br, dln_g, dln_b,
           W1, b1, W2, b2):
    g = _gates(y, norm_g.reshape(1, H), norm_b.reshape(1, H),
               Wl, bl.reshape(1, H), Wr, br.reshape(1, H))
    ei3 = edge_index.reshape(2, E // CB, CB)
    h = _sc_aggregate(g, ei3)
    out = _mlp(h, dln_g.reshape(2, H), dln_b.reshape(2, H),
               W1[:H], W1[H:], b1.reshape(1, H), W2, b2.reshape(1, N_OUT))
    return out


# speedup vs baseline: 3.8201x; 2.2855x over previous
import jax
import jax.numpy as jnp
from jax import lax
from jax.experimental import pallas as pl
from jax.experimental.pallas import tpu as pltpu
from jax.experimental.pallas import tpu_sc as plsc

N_NODES = 10000
E = 320000
H = 128
N_OUT = 128
_EPS = 1e-5

NC = 2
NS = 16
L = 16

N_PAD = 10240
NPT = N_PAD // NS
SUB = 80
CB = 80
K = 8
CH = CB * K
CHUNKS = E // CH

EB = 1280


def _gates_body(y_ref, ng_ref, nb_ref, wl_ref, bl_ref, wr_ref, br_ref, g_ref):
    y = y_ref[...]
    m = jnp.mean(y, axis=-1, keepdims=True)
    c = y - m
    v = jnp.mean(c * c, axis=-1, keepdims=True)
    yn = c * lax.rsqrt(v + _EPS) * ng_ref[...] + nb_ref[...]
    zl = jnp.dot(yn, wl_ref[...], preferred_element_type=jnp.float32) + bl_ref[...]
    zr = jnp.dot(yn, wr_ref[...], preferred_element_type=jnp.float32) + br_ref[...]
    g_ref[0] = 1.0 / (1.0 + jnp.exp(-zl))
    g_ref[1] = 1.0 / (1.0 + jnp.exp(-zr))


def _gates(y, ng, nb, wl, bl, wr, br):
    full = lambda shape: pl.BlockSpec(shape, lambda i: tuple(0 for _ in shape))
    return pl.pallas_call(
        _gates_body,
        grid=(E // EB,),
        in_specs=[
            pl.BlockSpec((EB, H), lambda i: (i, 0)),
            full((1, H)), full((1, H)),
            full((H, H)), full((1, H)),
            full((H, H)), full((1, H)),
        ],
        out_specs=pl.BlockSpec((2, EB, H), lambda i: (0, i, 0)),
        out_shape=jax.ShapeDtypeStruct((2, E, H), jnp.float32),
    )(y, ng, nb, wl, bl, wr, br)


def _sc_body(g_hbm, ei_hbm, h_hbm, acc, degw, rA, rB, idxb, nbuf, dslab,
             onesb, sem, sem2):
    cid = lax.axis_index("c")
    sid = lax.axis_index("s")
    node0 = sid * NPT
    n_my = jnp.where(sid < CHUNKS % NS, CHUNKS // NS + 1, CHUNKS // NS)
    zero16 = jnp.zeros((L,), jnp.float32)
    one16 = jnp.ones((L,), jnp.float32)
    bufs = (rA, rB)

    def _zrow(i, _):
        def _zcol(j, _):
            nbuf[i, pl.ds(j * L, L)] = zero16
            return 0
        return lax.fori_loop(0, H // L, _zcol, 0)
    lax.fori_loop(0, SUB, _zrow, 0)

    def _zd(i, _):
        dslab[i, pl.ds(0, L)] = zero16
        onesb[i, pl.ds(0, L)] = one16
        return 0
    lax.fori_loop(0, CB, _zd, 0)

    for k in range(NPT // SUB):
        pltpu.sync_copy(nbuf, acc.at[pl.ds(node0 + k * SUB, SUB)])
        pltpu.sync_copy(dslab, degw.at[pl.ds(node0 + k * SUB, SUB)])
    plsc.subcore_barrier()

    def _scatter_chunk(t, _):
        c = sid + NS * t
        pltpu.sync_copy(ei_hbm.at[1, pl.ds(c * K, K)], idxb)
        e0 = c * CH
        d = {0: pltpu.async_copy(g_hbm.at[cid, pl.ds(e0, CB)], bufs[0], sem)}
        for j in range(K):
            b = j % 2
            d[b].wait()
            if j + 1 < K:
                d[1 - b] = pltpu.async_copy(
                    g_hbm.at[cid, pl.ds(e0 + (j + 1) * CB, CB)], bufs[1 - b],
                    sem)
            pltpu.sync_copy(bufs[b], acc.at[idxb.at[j]], add=True)
            pltpu.sync_copy(onesb, degw.at[idxb.at[j]], add=True)
        return 0
    lax.fori_loop(0, n_my, _scatter_chunk, 0)
    plsc.subcore_barrier()

    for k in range(NPT // SUB):
        base = node0 + k * SUB
        pltpu.sync_copy(acc.at[pl.ds(base, SUB)], nbuf)
        pltpu.sync_copy(degw.at[pl.ds(base, SUB)], dslab)

        def _nrow(n, _):
            deg = dslab[n, pl.ds(0, L)]
            inv = 1.0 / jnp.maximum(deg, 1.0)
            for j in range(H // L):
                nbuf[n, pl.ds(j * L, L)] = nbuf[n, pl.ds(j * L, L)] * inv
            return 0
        lax.fori_loop(0, SUB, _nrow, 0)
        pltpu.sync_copy(nbuf, acc.at[pl.ds(base, SUB)])
    plsc.subcore_barrier()

    def _gather_chunk(t, _):
        c = sid + NS * t
        pltpu.sync_copy(ei_hbm.at[cid, pl.ds(c * K, K)], idxb)
        e0 = c * CH
        gd = {0: pltpu.async_copy(acc.at[idxb.at[0]], bufs[0], sem)}
        wd = {}
        for j in range(K):
            b = j % 2
            gd[b].wait()
            if j + 1 < K:
                if (1 - b) in wd:
                    wd[1 - b].wait()
                gd[1 - b] = pltpu.async_copy(acc.at[idxb.at[j + 1]],
                                             bufs[1 - b], sem)
            wd[b] = pltpu.async_copy(
                bufs[b], h_hbm.at[cid, pl.ds(e0 + j * CB, CB)], sem2)
        wd[0].wait()
        wd[1].wait()
        return 0
    lax.fori_loop(0, n_my, _gather_chunk, 0)


def _sc_aggregate(g, ei3):
    mesh = plsc.VectorSubcoreMesh(core_axis_name="c", subcore_axis_name="s",
                                  num_cores=NC, num_subcores=NS)
    return pl.kernel(
        _sc_body,
        out_type=jax.ShapeDtypeStruct((2, E, H), jnp.float32),
        mesh=mesh,
        scratch_types=[
            pltpu.VMEM_SHARED((N_PAD, H), jnp.float32),
            pltpu.VMEM_SHARED((N_PAD, L), jnp.float32),
            pltpu.VMEM((CB, H), jnp.float32),
            pltpu.VMEM((CB, H), jnp.float32),
            pltpu.VMEM((K, CB), jnp.int32),
            pltpu.VMEM((SUB, H), jnp.float32),
            pltpu.VMEM((SUB, L), jnp.float32),
            pltpu.VMEM((CB, L), jnp.float32),
            pltpu.SemaphoreType.DMA,
            pltpu.SemaphoreType.DMA,
        ],
        compiler_params=pltpu.CompilerParams(needs_layout_passes=False,
                                             use_tc_tiling_on_sc=False),
    )(g, ei3)


def _mlp_body(h_ref, dg_ref, db_ref, w1l_ref, w1r_ref, b1_ref, w2_ref,
              b2_ref, o_ref):
    hl = h_ref[0]
    hr = h_ref[1]
    m = (jnp.sum(hl, axis=-1, keepdims=True)
         + jnp.sum(hr, axis=-1, keepdims=True)) / (2.0 * H)
    cl = hl - m
    cr = hr - m
    v = (jnp.sum(cl * cl, axis=-1, keepdims=True)
         + jnp.sum(cr * cr, axis=-1, keepdims=True)) / (2.0 * H)
    rstd = lax.rsqrt(v + _EPS)
    lnl = cl * rstd * dg_ref[0] + db_ref[0]
    lnr = cr * rstd * dg_ref[1] + db_ref[1]
    z = (jnp.dot(lnl, w1l_ref[...], preferred_element_type=jnp.float32)
         + jnp.dot(lnr, w1r_ref[...], preferred_element_type=jnp.float32)
         + b1_ref[...])
    e = jnp.exp(jnp.minimum(z, 0.0)) - 1.0
    a = jnp.where(z > 0.0, z, e)
    o_ref[...] = (jnp.dot(a, w2_ref[...], preferred_element_type=jnp.float32)
                  + b2_ref[...])


def _mlp(h, dg, db, w1l, w1r, b1, w2, b2):
    full = lambda shape: pl.BlockSpec(shape, lambda i: tuple(0 for _ in shape))
    return pl.pallas_call(
        _mlp_body,
        grid=(E // EB,),
        in_specs=[
            pl.BlockSpec((2, EB, H), lambda i: (0, i, 0)),
            full((2, H)), full((2, H)),
            full((H, H)), full((H, H)), full((1, H)),
            full((H, N_OUT)), full((1, N_OUT)),
        ],
        out_specs=pl.BlockSpec((EB, N_OUT), lambda i: (i, 0)),
        out_shape=jax.ShapeDtypeStruct((E, N_OUT), jnp.float32),
    )(h, dg, db, w1l, w1r, b1, w2, b2)


def kernel(y, edge_index, norm_g, norm_b, Wl, bl, Wr, br, dln_g, dln_b,
           W1, b1, W2, b2):
    g = _gates(y, norm_g.reshape(1, H), norm_b.reshape(1, H),
               Wl, bl.reshape(1, H), Wr, br.reshape(1, H))
    ei3 = edge_index.reshape(2, E // CB, CB)
    h = _sc_aggregate(g, ei3)
    out = _mlp(h, dln_g.reshape(2, H), dln_b.reshape(2, H),
               W1[:H], W1[H:], b1.reshape(1, H), W2, b2.reshape(1, N_OUT))
    return out


# speedup vs baseline: 4.5982x; 1.2037x over previous
import functools

import jax
import jax.numpy as jnp
from jax import lax
from jax.experimental import pallas as pl
from jax.experimental.pallas import tpu as pltpu
from jax.experimental.pallas import tpu_sc as plsc

N_NODES = 10000
E = 320000
H = 128
N_OUT = 128
_EPS = 1e-5

NC = 2
NS = 16
L = 16

N_PAD = 10240
NPT = N_PAD // NS
SUB = 80
CB = 80
K = 8
CH = CB * K

EH = E // 2
CHUNKS_H = EH // CH
ROWS_H = EH // CB

EBH = 3200
BLK_H = EH // EBH

_SC_PARAMS = pltpu.CompilerParams(needs_layout_passes=False,
                                  use_tc_tiling_on_sc=False)


def _mesh():
    return plsc.VectorSubcoreMesh(core_axis_name="c", subcore_axis_name="s",
                                  num_cores=NC, num_subcores=NS)


def _gates_body(y_ref, ng_ref, nb_ref, wl_ref, bl_ref, wr_ref, br_ref, g_ref):
    y = y_ref[...]
    m = jnp.mean(y, axis=-1, keepdims=True)
    c = y - m
    v = jnp.mean(c * c, axis=-1, keepdims=True)
    yn = c * lax.rsqrt(v + _EPS) * ng_ref[...] + nb_ref[...]
    zl = jnp.dot(yn, wl_ref[...], preferred_element_type=jnp.float32) + bl_ref[...]
    zr = jnp.dot(yn, wr_ref[...], preferred_element_type=jnp.float32) + br_ref[...]
    g_ref[0] = 1.0 / (1.0 + jnp.exp(-zl))
    g_ref[1] = 1.0 / (1.0 + jnp.exp(-zr))


def _gates_half(off_blk, y, ng, nb, wl, bl, wr, br):
    full = lambda shape: pl.BlockSpec(shape, lambda i: tuple(0 for _ in shape))
    return pl.pallas_call(
        _gates_body,
        grid=(BLK_H,),
        in_specs=[
            pl.BlockSpec((EBH, H), lambda i: (i + off_blk, 0)),
            full((1, H)), full((1, H)),
            full((H, H)), full((1, H)),
            full((H, H)), full((1, H)),
        ],
        out_specs=pl.BlockSpec((2, EBH, H), lambda i: (0, i, 0)),
        out_shape=jax.ShapeDtypeStruct((2, EH, H), jnp.float32),
    )(y, ng, nb, wl, bl, wr, br)


def _n_my(sid):
    return jnp.where(sid < CHUNKS_H % NS, CHUNKS_H // NS + 1, CHUNKS_H // NS)


def _zero_init(nbuf, dslab, onesb, acc, degw, node0):
    zero16 = jnp.zeros((L,), jnp.float32)
    one16 = jnp.ones((L,), jnp.float32)

    def _zrow(i, _):
        def _zcol(j, _):
            nbuf[i, pl.ds(j * L, L)] = zero16
            return 0
        return lax.fori_loop(0, H // L, _zcol, 0)
    lax.fori_loop(0, SUB, _zrow, 0)

    def _zd(i, _):
        dslab[i, pl.ds(0, L)] = zero16
        onesb[i, pl.ds(0, L)] = one16
        return 0
    lax.fori_loop(0, CB, _zd, 0)

    for k in range(NPT // SUB):
        pltpu.sync_copy(nbuf, acc.at[pl.ds(node0 + k * SUB, SUB)])
        pltpu.sync_copy(dslab, degw.at[pl.ds(node0 + k * SUB, SUB)])


def _sc_scatter_body(off_row, g_hbm, ei_hbm, accs_out, degs_out, acc, degw,
                     rA, rB, idxb, nbuf, dslab, onesb, sem, sem2, sem3):
    cid = lax.axis_index("c")
    sid = lax.axis_index("s")
    node0 = sid * NPT
    bufs = (rA, rB)

    _zero_init(nbuf, dslab, onesb, acc, degw, node0)
    plsc.subcore_barrier()

    def _scatter_chunk(t, _):
        c = sid + NS * t
        pltpu.sync_copy(ei_hbm.at[1, pl.ds(off_row + c * K, K)], idxb)
        e0 = c * CH
        d = {0: pltpu.async_copy(g_hbm.at[cid, pl.ds(e0, CB)], bufs[0], sem)}
        sd = {}
        od = {}
        for j in range(K):
            b = j % 2
            d[b].wait()
            if j + 1 < K:
                if (1 - b) in sd:
                    sd[1 - b].wait()
                    od[1 - b].wait()
                d[1 - b] = pltpu.async_copy(
                    g_hbm.at[cid, pl.ds(e0 + (j + 1) * CB, CB)], bufs[1 - b],
                    sem)
            sd[b] = pltpu.async_copy(bufs[b], acc.at[idxb.at[j]], sem2,
                                     add=True)
            od[b] = pltpu.async_copy(onesb, degw.at[idxb.at[j]], sem3,
                                     add=True)
        sd[0].wait()
        sd[1].wait()
        od[0].wait()
        od[1].wait()
        return 0
    lax.fori_loop(0, _n_my(sid), _scatter_chunk, 0)
    plsc.subcore_barrier()

    for k in range(NPT // SUB):
        base = node0 + k * SUB
        pltpu.sync_copy(acc.at[pl.ds(base, SUB)], nbuf)
        pltpu.sync_copy(nbuf, accs_out.at[cid, pl.ds(base, SUB)])
        pltpu.sync_copy(degw.at[pl.ds(base, SUB)], dslab)
        pltpu.sync_copy(dslab, degs_out.at[cid, pl.ds(base, SUB)])


def _sc_scatter(off_row, g, ei3):
    return pl.kernel(
        functools.partial(_sc_scatter_body, off_row),
        out_type=(jax.ShapeDtypeStruct((NC, N_PAD, H), jnp.float32),
                  jax.ShapeDtypeStruct((NC, N_PAD, L), jnp.float32)),
        mesh=_mesh(),
        scratch_types=[
            pltpu.VMEM_SHARED((N_PAD, H), jnp.float32),
            pltpu.VMEM_SHARED((N_PAD, L), jnp.float32),
            pltpu.VMEM((CB, H), jnp.float32),
            pltpu.VMEM((CB, H), jnp.float32),
            pltpu.VMEM((K, CB), jnp.int32),
            pltpu.VMEM((SUB, H), jnp.float32),
            pltpu.VMEM((SUB, L), jnp.float32),
            pltpu.VMEM((CB, L), jnp.float32),
            pltpu.SemaphoreType.DMA,
            pltpu.SemaphoreType.DMA,
            pltpu.SemaphoreType.DMA,
        ],
        compiler_params=_SC_PARAMS,
    )(g, ei3)


def _sc_gather_body(off_row, accsA, accsB, degsA, degsB, ei_hbm, h_hbm, acc,
                    rA, rB, idxb, nbuf, dslab, onesb, sem, sem2):
    cid = lax.axis_index("c")
    sid = lax.axis_index("s")
    node0 = sid * NPT
    bufs = (rA, rB)

    for k in range(NPT // SUB):
        base = node0 + k * SUB
        pltpu.sync_copy(accsA.at[cid, pl.ds(base, SUB)], nbuf)
        pltpu.sync_copy(accsB.at[cid, pl.ds(base, SUB)], rA)
        pltpu.sync_copy(degsA.at[cid, pl.ds(base, SUB)], dslab)
        pltpu.sync_copy(degsB.at[cid, pl.ds(base, SUB)], onesb)

        def _nrow(n, _):
            deg = dslab[n, pl.ds(0, L)] + onesb[n, pl.ds(0, L)]
            inv = 1.0 / jnp.maximum(deg, 1.0)
            for j in range(H // L):
                s = nbuf[n, pl.ds(j * L, L)] + rA[n, pl.ds(j * L, L)]
                nbuf[n, pl.ds(j * L, L)] = s * inv
            return 0
        lax.fori_loop(0, SUB, _nrow, 0)
        pltpu.sync_copy(nbuf, acc.at[pl.ds(base, SUB)])
    plsc.subcore_barrier()

    def _gather_chunk(t, _):
        c = sid + NS * t
        pltpu.sync_copy(ei_hbm.at[cid, pl.ds(off_row + c * K, K)], idxb)
        e0 = c * CH
        gd = {0: pltpu.async_copy(acc.at[idxb.at[0]], bufs[0], sem)}
        wd = {}
        for j in range(K):
            b = j % 2
            gd[b].wait()
            if j + 1 < K:
                if (1 - b) in wd:
                    wd[1 - b].wait()
                gd[1 - b] = pltpu.async_copy(acc.at[idxb.at[j + 1]],
                                             bufs[1 - b], sem)
            wd[b] = pltpu.async_copy(
                bufs[b], h_hbm.at[cid, pl.ds(e0 + j * CB, CB)], sem2)
        wd[0].wait()
        wd[1].wait()
        return 0
    lax.fori_loop(0, _n_my(sid), _gather_chunk, 0)


def _sc_gather(off_row, ei3, accsA, accsB, degsA, degsB):
    return pl.kernel(
        functools.partial(_sc_gather_body, off_row),
        out_type=jax.ShapeDtypeStruct((NC, EH, H), jnp.float32),
        mesh=_mesh(),
        scratch_types=[
            pltpu.VMEM_SHARED((N_PAD, H), jnp.float32),
            pltpu.VMEM((CB, H), jnp.float32),
            pltpu.VMEM((CB, H), jnp.float32),
            pltpu.VMEM((K, CB), jnp.int32),
            pltpu.VMEM((SUB, H), jnp.float32),
            pltpu.VMEM((SUB, L), jnp.float32),
            pltpu.VMEM((CB, L), jnp.float32),
            pltpu.SemaphoreType.DMA,
            pltpu.SemaphoreType.DMA,
        ],
        compiler_params=_SC_PARAMS,
    )(accsA, accsB, degsA, degsB, ei3)


def _mlp_body(h_ref, dg_ref, db_ref, w1l_ref, w1r_ref, b1_ref, w2_ref,
              b2_ref, o_ref):
    hl = h_ref[0]
    hr = h_ref[1]
    m = (jnp.sum(hl, axis=-1, keepdims=True)
         + jnp.sum(hr, axis=-1, keepdims=True)) / (2.0 * H)
    cl = hl - m
    cr = hr - m
    v = (jnp.sum(cl * cl, axis=-1, keepdims=True)
         + jnp.sum(cr * cr, axis=-1, keepdims=True)) / (2.0 * H)
    rstd = lax.rsqrt(v + _EPS)
    lnl = cl * rstd * dg_ref[0] + db_ref[0]
    lnr = cr * rstd * dg_ref[1] + db_ref[1]
    z = (jnp.dot(lnl, w1l_ref[...], preferred_element_type=jnp.float32)
         + jnp.dot(lnr, w1r_ref[...], preferred_element_type=jnp.float32)
         + b1_ref[...])
    e = jnp.exp(jnp.minimum(z, 0.0)) - 1.0
    a = jnp.where(z > 0.0, z, e)
    o_ref[...] = (jnp.dot(a, w2_ref[...], preferred_element_type=jnp.float32)
                  + b2_ref[...])


def _mlp_body_aliased(prev_ref, *rest):
    del prev_ref
    _mlp_body(*rest)


def _mlp_half(off_blk, h, dg, db, w1l, w1r, b1, w2, b2, prev=None):
    full = lambda shape: pl.BlockSpec(shape, lambda i: tuple(0 for _ in shape))
    specs = [
        pl.BlockSpec((2, EBH, H), lambda i: (0, i, 0)),
        full((2, H)), full((2, H)),
        full((H, H)), full((H, H)), full((1, H)),
        full((H, N_OUT)), full((1, N_OUT)),
    ]
    args = (h, dg, db, w1l, w1r, b1, w2, b2)
    body = _mlp_body
    aliases = {}
    if prev is not None:
        specs = [pl.BlockSpec(memory_space=pl.ANY)] + specs
        args = (prev,) + args
        body = _mlp_body_aliased
        aliases = {0: 0}
    return pl.pallas_call(
        body,
        grid=(BLK_H,),
        in_specs=specs,
        out_specs=pl.BlockSpec((EBH, N_OUT), lambda i: (i + off_blk, 0)),
        out_shape=jax.ShapeDtypeStruct((E, N_OUT), jnp.float32),
        input_output_aliases=aliases,
    )(*args)


def kernel(y, edge_index, norm_g, norm_b, Wl, bl, Wr, br, dln_g, dln_b,
           W1, b1, W2, b2):
    ng = norm_g.reshape(1, H)
    nb = norm_b.reshape(1, H)
    blr = bl.reshape(1, H)
    brr = br.reshape(1, H)
    ei3 = edge_index.reshape(2, E // CB, CB)

    gA = _gates_half(0, y, ng, nb, Wl, blr, Wr, brr)
    gB = _gates_half(BLK_H, y, ng, nb, Wl, blr, Wr, brr)
    accsA, degsA = _sc_scatter(0, gA, ei3)
    accsB, degsB = _sc_scatter(ROWS_H, gB, ei3)
    hA = _sc_gather(0, ei3, accsA, accsB, degsA, degsB)
    hB = _sc_gather(ROWS_H, ei3, accsA, accsB, degsA, degsB)

    dg = dln_g.reshape(2, H)
    db = dln_b.reshape(2, H)
    b1r = b1.reshape(1, H)
    b2r = b2.reshape(1, N_OUT)
    outA = _mlp_half(0, hA, dg, db, W1[:H], W1[H:], b1r, W2, b2r)
    out = _mlp_half(BLK_H, hB, dg, db, W1[:H], W1[H:], b1r, W2, b2r,
                    prev=outA)
    return out


# speedup vs baseline: 4.8474x; 1.0542x over previous
import functools

import jax
import jax.numpy as jnp
from jax import lax
from jax.experimental import pallas as pl
from jax.experimental.pallas import tpu as pltpu
from jax.experimental.pallas import tpu_sc as plsc

N_NODES = 10000
E = 320000
H = 128
N_OUT = 128
_EPS = 1e-5

NC = 2
NS = 16
L = 16

N_PAD = 10240
NPT = N_PAD // NS
SUB = 80
CB = 80
K = 8
CH = CB * K

EH = E // 2
CHUNKS_H = EH // CH
ROWS_H = EH // CB

EBH = 3200
BLK_H = EH // EBH

_SC_PARAMS = pltpu.CompilerParams(needs_layout_passes=False,
                                  use_tc_tiling_on_sc=False)


def _mesh():
    return plsc.VectorSubcoreMesh(core_axis_name="c", subcore_axis_name="s",
                                  num_cores=NC, num_subcores=NS)


def _gates_body(y_ref, ng_ref, nb_ref, wl_ref, bl_ref, wr_ref, br_ref, g_ref):
    y = y_ref[...]
    m = jnp.mean(y, axis=-1, keepdims=True)
    c = y - m
    v = jnp.mean(c * c, axis=-1, keepdims=True)
    yn = c * lax.rsqrt(v + _EPS) * ng_ref[...] + nb_ref[...]
    zl = jnp.dot(yn, wl_ref[...], preferred_element_type=jnp.float32) + bl_ref[...]
    zr = jnp.dot(yn, wr_ref[...], preferred_element_type=jnp.float32) + br_ref[...]
    g_ref[0] = 1.0 / (1.0 + jnp.exp(-zl))
    g_ref[1] = 1.0 / (1.0 + jnp.exp(-zr))


def _gates_half(off_blk, y, ng, nb, wl, bl, wr, br):
    full = lambda shape: pl.BlockSpec(shape, lambda i: tuple(0 for _ in shape))
    return pl.pallas_call(
        _gates_body,
        grid=(BLK_H,),
        in_specs=[
            pl.BlockSpec((EBH, H), lambda i: (i + off_blk, 0)),
            full((1, H)), full((1, H)),
            full((H, H)), full((1, H)),
            full((H, H)), full((1, H)),
        ],
        out_specs=pl.BlockSpec((2, EBH, H), lambda i: (0, i, 0)),
        out_shape=jax.ShapeDtypeStruct((2, EH, H), jnp.float32),
    )(y, ng, nb, wl, bl, wr, br)


def _n_my(sid):
    return jnp.where(sid < CHUNKS_H % NS, CHUNKS_H // NS + 1, CHUNKS_H // NS)


def _zero_init(nbuf, dslab, onesb, acc, degw, node0, sem):
    zero16 = jnp.zeros((L,), jnp.float32)
    one16 = jnp.ones((L,), jnp.float32)

    def _zrow(i, _):
        def _zcol(j, _):
            nbuf[i, pl.ds(j * L, L)] = zero16
            return 0
        return lax.fori_loop(0, H // L, _zcol, 0)
    lax.fori_loop(0, SUB, _zrow, 0)

    def _zd(i, _):
        dslab[i, pl.ds(0, L)] = zero16
        onesb[i, pl.ds(0, L)] = one16
        return 0
    lax.fori_loop(0, CB, _zd, 0)

    zd = []
    for k in range(NPT // SUB):
        zd.append(pltpu.async_copy(nbuf, acc.at[pl.ds(node0 + k * SUB, SUB)],
                                   sem))
        zd.append(pltpu.async_copy(dslab,
                                   degw.at[pl.ds(node0 + k * SUB, SUB)], sem))
    for d in zd:
        d.wait()


def _sc_scatter_body(off_row, g_hbm, ei_hbm, accs_out, degs_out, acc, degw,
                     rA, rB, idxb, nbuf, dslab, onesb, sem, sem2, sem3):
    cid = lax.axis_index("c")
    sid = lax.axis_index("s")
    node0 = sid * NPT
    bufs = (rA, rB)

    _zero_init(nbuf, dslab, onesb, acc, degw, node0, sem)
    plsc.subcore_barrier()

    def _scatter_chunk(t, _):
        c = sid + NS * t
        pltpu.sync_copy(ei_hbm.at[1, pl.ds(off_row + c * K, K)], idxb)
        e0 = c * CH
        d = {0: pltpu.async_copy(g_hbm.at[cid, pl.ds(e0, CB)], bufs[0], sem)}
        sd = {}
        od = {}
        for j in range(K):
            b = j % 2
            d[b].wait()
            if j + 1 < K:
                if (1 - b) in sd:
                    sd[1 - b].wait()
                    od[1 - b].wait()
                d[1 - b] = pltpu.async_copy(
                    g_hbm.at[cid, pl.ds(e0 + (j + 1) * CB, CB)], bufs[1 - b],
                    sem)
            sd[b] = pltpu.async_copy(bufs[b], acc.at[idxb.at[j]], sem2,
                                     add=True)
            od[b] = pltpu.async_copy(onesb, degw.at[idxb.at[j]], sem3,
                                     add=True)
        sd[0].wait()
        sd[1].wait()
        od[0].wait()
        od[1].wait()
        return 0
    lax.fori_loop(0, _n_my(sid), _scatter_chunk, 0)
    plsc.subcore_barrier()

    ed = [
        pltpu.async_copy(acc.at[pl.ds(node0, NPT)],
                         accs_out.at[cid, pl.ds(node0, NPT)], sem),
        pltpu.async_copy(degw.at[pl.ds(node0, NPT)],
                         degs_out.at[cid, pl.ds(node0, NPT)], sem2),
    ]
    for d in ed:
        d.wait()


def _sc_scatter(off_row, g, ei3):
    return pl.kernel(
        functools.partial(_sc_scatter_body, off_row),
        out_type=(jax.ShapeDtypeStruct((NC, N_PAD, H), jnp.float32),
                  jax.ShapeDtypeStruct((NC, N_PAD, L), jnp.float32)),
        mesh=_mesh(),
        scratch_types=[
            pltpu.VMEM_SHARED((N_PAD, H), jnp.float32),
            pltpu.VMEM_SHARED((N_PAD, L), jnp.float32),
            pltpu.VMEM((CB, H), jnp.float32),
            pltpu.VMEM((CB, H), jnp.float32),
            pltpu.VMEM((K, CB), jnp.int32),
            pltpu.VMEM((SUB, H), jnp.float32),
            pltpu.VMEM((SUB, L), jnp.float32),
            pltpu.VMEM((CB, L), jnp.float32),
            pltpu.SemaphoreType.DMA,
            pltpu.SemaphoreType.DMA,
            pltpu.SemaphoreType.DMA,
        ],
        compiler_params=_SC_PARAMS,
    )(g, ei3)


def _sc_gather_body(off_row, accsA, accsB, degsA, degsB, ei_hbm, h_hbm, acc,
                    rA, rB, idxb, nbuf, nbuf2, dslab, dslab2, onesb, onesb2,
                    sem, sem2):
    cid = lax.axis_index("c")
    sid = lax.axis_index("s")
    node0 = sid * NPT
    bufs = (rA, rB)
    sets = ((nbuf, rA, dslab, onesb), (nbuf2, rB, dslab2, onesb2))
    NSC = NPT // SUB

    def _prefetch(k, b):
        base = node0 + k * SUB
        nb, ra, dl, ob = sets[b]
        return [
            pltpu.async_copy(accsA.at[cid, pl.ds(base, SUB)], nb, sem),
            pltpu.async_copy(accsB.at[cid, pl.ds(base, SUB)], ra, sem),
            pltpu.async_copy(degsA.at[cid, pl.ds(base, SUB)], dl, sem),
            pltpu.async_copy(degsB.at[cid, pl.ds(base, SUB)], ob, sem),
        ]

    pf = {0: _prefetch(0, 0)}
    wb = {}
    for k in range(NSC):
        b = k % 2
        for d in pf[b]:
            d.wait()
        if k + 1 < NSC:
            if (1 - b) in wb:
                wb[1 - b].wait()
            pf[1 - b] = _prefetch(k + 1, 1 - b)
        nb, ra, dl, ob = sets[b]

        def _nrow(n, _):
            deg = dl[n, pl.ds(0, L)] + ob[n, pl.ds(0, L)]
            inv = 1.0 / jnp.maximum(deg, 1.0)
            for j in range(H // L):
                s = nb[n, pl.ds(j * L, L)] + ra[n, pl.ds(j * L, L)]
                nb[n, pl.ds(j * L, L)] = s * inv
            return 0
        lax.fori_loop(0, SUB, _nrow, 0)
        wb[b] = pltpu.async_copy(nb, acc.at[pl.ds(node0 + k * SUB, SUB)],
                                 sem2)
    wb[0].wait()
    wb[1].wait()
    plsc.subcore_barrier()

    def _gather_chunk(t, _):
        c = sid + NS * t
        pltpu.sync_copy(ei_hbm.at[cid, pl.ds(off_row + c * K, K)], idxb)
        e0 = c * CH
        gd = {0: pltpu.async_copy(acc.at[idxb.at[0]], bufs[0], sem)}
        wd = {}
        for j in range(K):
            b = j % 2
            gd[b].wait()
            if j + 1 < K:
                if (1 - b) in wd:
                    wd[1 - b].wait()
                gd[1 - b] = pltpu.async_copy(acc.at[idxb.at[j + 1]],
                                             bufs[1 - b], sem)
            wd[b] = pltpu.async_copy(
                bufs[b], h_hbm.at[cid, pl.ds(e0 + j * CB, CB)], sem2)
        wd[0].wait()
        wd[1].wait()
        return 0
    lax.fori_loop(0, _n_my(sid), _gather_chunk, 0)


def _sc_gather(off_row, ei3, accsA, accsB, degsA, degsB):
    return pl.kernel(
        functools.partial(_sc_gather_body, off_row),
        out_type=jax.ShapeDtypeStruct((NC, EH, H), jnp.float32),
        mesh=_mesh(),
        scratch_types=[
            pltpu.VMEM_SHARED((N_PAD, H), jnp.float32),
            pltpu.VMEM((CB, H), jnp.float32),
            pltpu.VMEM((CB, H), jnp.float32),
            pltpu.VMEM((K, CB), jnp.int32),
            pltpu.VMEM((SUB, H), jnp.float32),
            pltpu.VMEM((SUB, H), jnp.float32),
            pltpu.VMEM((SUB, L), jnp.float32),
            pltpu.VMEM((SUB, L), jnp.float32),
            pltpu.VMEM((SUB, L), jnp.float32),
            pltpu.VMEM((SUB, L), jnp.float32),
            pltpu.SemaphoreType.DMA,
            pltpu.SemaphoreType.DMA,
        ],
        compiler_params=_SC_PARAMS,
    )(accsA, accsB, degsA, degsB, ei3)


def _mlp_body(h_ref, dg_ref, db_ref, w1l_ref, w1r_ref, b1_ref, w2_ref,
              b2_ref, o_ref):
    hl = h_ref[0]
    hr = h_ref[1]
    m = (jnp.sum(hl, axis=-1, keepdims=True)
         + jnp.sum(hr, axis=-1, keepdims=True)) / (2.0 * H)
    cl = hl - m
    cr = hr - m
    v = (jnp.sum(cl * cl, axis=-1, keepdims=True)
         + jnp.sum(cr * cr, axis=-1, keepdims=True)) / (2.0 * H)
    rstd = lax.rsqrt(v + _EPS)
    lnl = cl * rstd * dg_ref[0] + db_ref[0]
    lnr = cr * rstd * dg_ref[1] + db_ref[1]
    z = (jnp.dot(lnl, w1l_ref[...], preferred_element_type=jnp.float32)
         + jnp.dot(lnr, w1r_ref[...], preferred_element_type=jnp.float32)
         + b1_ref[...])
    e = jnp.exp(jnp.minimum(z, 0.0)) - 1.0
    a = jnp.where(z > 0.0, z, e)
    o_ref[...] = (jnp.dot(a, w2_ref[...], preferred_element_type=jnp.float32)
                  + b2_ref[...])


def _mlp_body_aliased(prev_ref, *rest):
    del prev_ref
    _mlp_body(*rest)


def _mlp_half(off_blk, h, dg, db, w1l, w1r, b1, w2, b2, prev=None):
    full = lambda shape: pl.BlockSpec(shape, lambda i: tuple(0 for _ in shape))
    specs = [
        pl.BlockSpec((2, EBH, H), lambda i: (0, i, 0)),
        full((2, H)), full((2, H)),
        full((H, H)), full((H, H)), full((1, H)),
        full((H, N_OUT)), full((1, N_OUT)),
    ]
    args = (h, dg, db, w1l, w1r, b1, w2, b2)
    body = _mlp_body
    aliases = {}
    if prev is not None:
        specs = [pl.BlockSpec(memory_space=pl.ANY)] + specs
        args = (prev,) + args
        body = _mlp_body_aliased
        aliases = {0: 0}
    return pl.pallas_call(
        body,
        grid=(BLK_H,),
        in_specs=specs,
        out_specs=pl.BlockSpec((EBH, N_OUT), lambda i: (i + off_blk, 0)),
        out_shape=jax.ShapeDtypeStruct((E, N_OUT), jnp.float32),
        input_output_aliases=aliases,
    )(*args)


def kernel(y, edge_index, norm_g, norm_b, Wl, bl, Wr, br, dln_g, dln_b,
           W1, b1, W2, b2):
    ng = norm_g.reshape(1, H)
    nb = norm_b.reshape(1, H)
    blr = bl.reshape(1, H)
    brr = br.reshape(1, H)
    ei3 = edge_index.reshape(2, E // CB, CB)

    gA = _gates_half(0, y, ng, nb, Wl, blr, Wr, brr)
    gB = _gates_half(BLK_H, y, ng, nb, Wl, blr, Wr, brr)
    accsA, degsA = _sc_scatter(0, gA, ei3)
    accsB, degsB = _sc_scatter(ROWS_H, gB, ei3)
    hA = _sc_gather(0, ei3, accsA, accsB, degsA, degsB)
    hB = _sc_gather(ROWS_H, ei3, accsA, accsB, degsA, degsB)

    dg = dln_g.reshape(2, H)
    db = dln_b.reshape(2, H)
    b1r = b1.reshape(1, H)
    b2r = b2.reshape(1, N_OUT)
    outA = _mlp_half(0, hA, dg, db, W1[:H], W1[H:], b1r, W2, b2r)
    out = _mlp_half(BLK_H, hB, dg, db, W1[:H], W1[H:], b1r, W2, b2r,
                    prev=outA)
    return out
